# Initial kernel scaffold; baseline (speedup 1.0000x reference)
import jax
import jax.numpy as jnp
from jax.experimental import pallas as pl


def kernel(x, weight):
    raise NotImplementedError("write your pallas kernel here")


# speedup vs baseline: 1.8421x; 1.8421x over previous
import functools

import jax
import jax.numpy as jnp
from jax import lax
from jax.experimental import pallas as pl
from jax.experimental.pallas import tpu as pltpu
from jax.experimental.pallas import tpu_sc as plsc

_B = 16384 * 50
_D = 64
_NW = 32
_GSZ = 128
_ROWS_PER_W = _B // _NW
_G_PER_W = _ROWS_PER_W // _GSZ
_K = 5
_CHUNKS = _G_PER_W // _K

_mesh = plsc.VectorSubcoreMesh(core_axis_name="c", subcore_axis_name="s")


@functools.partial(
    pl.kernel,
    mesh=_mesh,
    compiler_params=pltpu.CompilerParams(use_tc_tiling_on_sc=False),
    out_type=jax.ShapeDtypeStruct((_B // _GSZ, _GSZ, _D), jnp.float32),
    scratch_types=[
        pltpu.VMEM((_G_PER_W, _GSZ), jnp.int32),
        pltpu.VMEM((_K, _GSZ, _D), jnp.float32),
        pltpu.SemaphoreType.DMA,
    ],
)
def _emb_lookup(idx_hbm, table_hbm, out_hbm, idx_v, rows_v, sem):
    wid = lax.axis_index("s") * 2 + lax.axis_index("c")
    grow = wid * _G_PER_W
    pltpu.sync_copy(idx_hbm.at[pl.ds(grow, _G_PER_W)], idx_v)

    def body(c, _):
        descs = []
        for g in range(_K):
            descs.append(
                pltpu.async_copy(
                    table_hbm.at[idx_v.at[c * _K + g]],
                    rows_v.at[g],
                    sem,
                )
            )
        for d in descs:
            d.wait()
        pltpu.sync_copy(rows_v, out_hbm.at[pl.ds(grow + c * _K, _K)])
        return ()

    lax.fori_loop(0, _CHUNKS, body, (), unroll=False)


def kernel(x, weight):
    idx = x.reshape(_B // _GSZ, _GSZ).astype(jnp.int32)
    out = _emb_lookup(idx, weight)
    return out.reshape(x.shape[0], x.shape[1], _D)


# speedup vs baseline: 1.8740x; 1.0173x over previous
import functools

import jax
import jax.numpy as jnp
from jax import lax
from jax.experimental import pallas as pl
from jax.experimental.pallas import tpu as pltpu
from jax.experimental.pallas import tpu_sc as plsc

_B = 16384 * 50
_D = 64
_NW = 32
_GSZ = 128
_ROWS_PER_W = _B // _NW
_G_PER_W = _ROWS_PER_W // _GSZ
_K = 5
_CHUNKS = _G_PER_W // _K

_mesh = plsc.VectorSubcoreMesh(core_axis_name="c", subcore_axis_name="s")


@functools.partial(
    pl.kernel,
    mesh=_mesh,
    compiler_params=pltpu.CompilerParams(use_tc_tiling_on_sc=False),
    out_type=jax.ShapeDtypeStruct((_B // _GSZ, _GSZ, _D), jnp.float32),
    scratch_types=[
        pltpu.VMEM((_G_PER_W, _GSZ), jnp.int32),
        pltpu.VMEM((2, _K, _GSZ, _D), jnp.float32),
        pltpu.SemaphoreType.DMA,
        pltpu.SemaphoreType.DMA,
    ],
)
def _emb_lookup(idx_hbm, table_hbm, out_hbm, idx_v, rows_v, sem0, sem1):
    wid = lax.axis_index("s") * 2 + lax.axis_index("c")
    grow = wid * _G_PER_W
    pltpu.sync_copy(idx_hbm.at[pl.ds(grow, _G_PER_W)], idx_v)
    sems = (sem0, sem1)

    def fire(c, b):
        for g in range(_K):
            pltpu.async_copy(
                table_hbm.at[idx_v.at[c * _K + g]], rows_v.at[b, g], sems[b]
            )

    def drain(b):
        pltpu.make_async_copy(
            out_hbm.at[pl.ds(0, _K)], rows_v.at[b], sems[b]
        ).wait()

    def flush(c, b):
        pltpu.sync_copy(rows_v.at[b], out_hbm.at[pl.ds(grow + c * _K, _K)])

    fire(0, 0)

    def body(j, _):
        c0 = j * 2
        drain(0)
        fire(c0 + 1, 1)
        flush(c0, 0)
        drain(1)

        @pl.when(c0 + 2 < _CHUNKS)
        def _():
            fire(c0 + 2, 0)

        flush(c0 + 1, 1)
        return ()

    lax.fori_loop(0, _CHUNKS // 2, body, (), unroll=False)


def kernel(x, weight):
    idx = x.reshape(_B // _GSZ, _GSZ).astype(jnp.int32)
    out = _emb_lookup(idx, weight)
    return out.reshape(x.shape[0], x.shape[1], _D)
